# baseline (speedup 1.0000x reference)
import functools

import jax
import jax.numpy as jnp
from jax import lax
from jax.experimental import pallas as pl
from jax.experimental.pallas import tpu as pltpu
from jax.experimental.pallas import tpu_sc as plsc

_info = plsc.get_sparse_core_info()
_NC, _NS = _info.num_cores, _info.num_subcores
_NW = _NC * _NS
_L = 128


_TBLK = 2048


@jax.jit
def _tc_transpose(weight_t):
    dim, nrows = weight_t.shape
    grid = (nrows + _TBLK - 1) // _TBLK

    def body(in_ref, out_ref):
        out_ref[...] = in_ref[...].T

    return pl.pallas_call(
        body,
        grid=(grid,),
        in_specs=[pl.BlockSpec((dim, _TBLK), lambda i: (0, i))],
        out_specs=pl.BlockSpec((_TBLK, dim), lambda i: (i, 0)),
        out_shape=jax.ShapeDtypeStruct((nrows, dim), jnp.float32),
    )(weight_t)


@jax.jit
def _sc_gather(weight, idx_t):
    nrows, dim = weight.shape
    sl, nb = idx_t.shape

    mesh = plsc.VectorSubcoreMesh(core_axis_name="c", subcore_axis_name="s")

    @functools.partial(
        pl.kernel,
        out_type=jax.ShapeDtypeStruct((sl, dim, nb), jnp.float32),
        mesh=mesh,
        scratch_types=[
            pltpu.VMEM((sl, _L), jnp.int32),
            pltpu.VMEM((2, _L, dim), jnp.float32),
            pltpu.VMEM((2, 1, dim, _L), jnp.float32),
            pltpu.SemaphoreType.DMA((2,)),
            pltpu.SemaphoreType.DMA((2,)),
        ],
        compiler_params=pltpu.CompilerParams(use_tc_tiling_on_sc=True, needs_layout_passes=False),
    )
    def k(table_hbm, idx_hbm, out_hbm, idx_v, rows_v, blk_v, gsem, wsem):
        wid = lax.axis_index("s") * _NC + lax.axis_index("c")
        lane0 = pl.multiple_of(wid * _L, _L)

        pltpu.sync_copy(idx_hbm.at[:, pl.ds(lane0, _L)], idx_v)

        def fire_rows(s, p):
            vecs = [idx_v.at[s][pl.ds(16 * g, 16)] for g in range(_L // 16)]
            for b in range(_L):
                iv = vecs[b // 16][b % 16]
                pltpu.async_copy(
                    table_hbm.at[pl.ds(iv, 1)],
                    rows_v.at[p].at[pl.ds(b, 1)],
                    gsem.at[p],
                )

        def drain_rows(p):
            pltpu.make_async_copy(
                table_hbm.at[pl.ds(0, _L)], rows_v.at[p], gsem.at[p]
            ).wait()

        def transpose(p):
            src = rows_v.at[p]
            dst = blk_v.at[p, 0]
            for g in range(_L // 16):
                bvec = jax.lax.iota(jnp.int32, 16) + 16 * g
                for d in range(dim):
                    dvec = jnp.full((16,), d, jnp.int32)
                    col = plsc.load_gather(src, [bvec, dvec])
                    dst[d, pl.ds(16 * g, 16)] = col

        def fire_write(s, p):
            pltpu.async_copy(
                blk_v.at[p],
                out_hbm.at[pl.ds(s, 1), :, pl.ds(lane0, _L)],
                wsem.at[p],
            )

        def drain_write(s, p):
            pltpu.make_async_copy(
                blk_v.at[p],
                out_hbm.at[pl.ds(s, 1), :, pl.ds(lane0, _L)],
                wsem.at[p],
            ).wait()

        fire_rows(0, 0)

        @pl.loop(1, sl)
        def _(s):
            p = lax.rem(s, 2)
            @pl.when(s >= 3)
            def _():
                drain_write(s - 3, 1 - p)
            fire_rows(s, p)
            drain_rows(1 - p)
            transpose(1 - p)
            fire_write(s - 1, 1 - p)

        pf = lax.rem(sl - 1, 2)
        drain_rows(pf)
        drain_write(sl - 3, pf)
        transpose(pf)
        fire_write(sl - 1, pf)
        drain_write(sl - 2, 1 - pf)
        drain_write(sl - 1, pf)

    return k(weight, idx_t)


def kernel(idx, weight):
    out_t = _sc_gather(_tc_transpose(weight.T), idx.T)
    return jnp.transpose(out_t, (2, 0, 1))


# speedup vs baseline: 1.4467x; 1.4467x over previous
import functools

import jax
import jax.numpy as jnp
from jax import lax
from jax.experimental import pallas as pl
from jax.experimental.pallas import tpu as pltpu
from jax.experimental.pallas import tpu_sc as plsc

_info = plsc.get_sparse_core_info()
_NC, _NS = _info.num_cores, _info.num_subcores
_NW = _NC * _NS
_L = 128


_TBLK = 8192


@jax.jit
def _tc_transpose(weight_t):
    dim, nrows = weight_t.shape
    grid = (nrows + _TBLK - 1) // _TBLK

    def body(in_ref, out_ref):
        out_ref[...] = in_ref[...].T

    return pl.pallas_call(
        body,
        grid=(grid,),
        in_specs=[pl.BlockSpec((dim, _TBLK), lambda i: (0, i))],
        out_specs=pl.BlockSpec((_TBLK, dim), lambda i: (i, 0)),
        out_shape=jax.ShapeDtypeStruct((nrows, dim), jnp.float32),
    )(weight_t)


@jax.jit
def _sc_gather(weight, idx_t):
    nrows, dim = weight.shape
    sl, nb = idx_t.shape

    mesh = plsc.VectorSubcoreMesh(core_axis_name="c", subcore_axis_name="s")

    @functools.partial(
        pl.kernel,
        out_type=jax.ShapeDtypeStruct((sl, dim, nb), jnp.float32),
        mesh=mesh,
        scratch_types=[
            pltpu.VMEM((sl, _L), jnp.int32),
            pltpu.VMEM((2, _L, dim), jnp.float32),
            pltpu.VMEM((2, 1, dim, _L), jnp.float32),
            pltpu.SemaphoreType.DMA((2,)),
            pltpu.SemaphoreType.DMA((2,)),
        ],
        compiler_params=pltpu.CompilerParams(use_tc_tiling_on_sc=True, needs_layout_passes=False),
    )
    def k(table_hbm, idx_hbm, out_hbm, idx_v, rows_v, blk_v, gsem, wsem):
        wid = lax.axis_index("s") * _NC + lax.axis_index("c")
        lane0 = pl.multiple_of(wid * _L, _L)

        pltpu.sync_copy(idx_hbm.at[:, pl.ds(lane0, _L)], idx_v)

        def fire_rows(s, p):
            vecs = [idx_v.at[s][pl.ds(16 * g, 16)] for g in range(_L // 16)]
            for b in range(_L):
                iv = vecs[b // 16][b % 16]
                pltpu.async_copy(
                    table_hbm.at[pl.ds(iv, 1)],
                    rows_v.at[p].at[pl.ds(b, 1)],
                    gsem.at[p],
                )

        def drain_rows(p):
            pltpu.make_async_copy(
                table_hbm.at[pl.ds(0, _L)], rows_v.at[p], gsem.at[p]
            ).wait()

        def transpose(p):
            src = rows_v.at[p]
            dst = blk_v.at[p, 0]
            for g in range(_L // 16):
                bvec = jax.lax.iota(jnp.int32, 16) + 16 * g
                for d in range(dim):
                    dvec = jnp.full((16,), d, jnp.int32)
                    col = plsc.load_gather(src, [bvec, dvec])
                    dst[d, pl.ds(16 * g, 16)] = col

        def fire_write(s, p):
            pltpu.async_copy(
                blk_v.at[p],
                out_hbm.at[pl.ds(s, 1), :, pl.ds(lane0, _L)],
                wsem.at[p],
            )

        def drain_write(s, p):
            pltpu.make_async_copy(
                blk_v.at[p],
                out_hbm.at[pl.ds(s, 1), :, pl.ds(lane0, _L)],
                wsem.at[p],
            ).wait()

        fire_rows(0, 0)

        @pl.loop(1, sl)
        def _(s):
            p = lax.rem(s, 2)
            @pl.when(s >= 3)
            def _():
                drain_write(s - 3, 1 - p)
            fire_rows(s, p)
            drain_rows(1 - p)
            transpose(1 - p)
            fire_write(s - 1, 1 - p)

        pf = lax.rem(sl - 1, 2)
        drain_rows(pf)
        drain_write(sl - 3, pf)
        transpose(pf)
        fire_write(sl - 1, pf)
        drain_write(sl - 2, 1 - pf)
        drain_write(sl - 1, pf)

    return k(weight, idx_t)


def kernel(idx, weight):
    out_t = _sc_gather(_tc_transpose(weight.T), idx.T)
    return jnp.transpose(out_t, (2, 0, 1))


# speedup vs baseline: 1.5755x; 1.0890x over previous
import functools

import jax
import jax.numpy as jnp
from jax import lax
from jax.experimental import pallas as pl
from jax.experimental.pallas import tpu as pltpu
from jax.experimental.pallas import tpu_sc as plsc

_info = plsc.get_sparse_core_info()
_NC, _NS = _info.num_cores, _info.num_subcores
_NW = _NC * _NS
_L = 128


_TBLK = 40960


@jax.jit
def _tc_transpose(weight_t):
    dim, nrows = weight_t.shape
    grid = (nrows + _TBLK - 1) // _TBLK

    def body(in_ref, out_ref):
        out_ref[...] = in_ref[...].T

    return pl.pallas_call(
        body,
        grid=(grid,),
        in_specs=[pl.BlockSpec((dim, _TBLK), lambda i: (0, i))],
        out_specs=pl.BlockSpec((_TBLK, dim), lambda i: (i, 0)),
        out_shape=jax.ShapeDtypeStruct((nrows, dim), jnp.float32),
    )(weight_t)


@jax.jit
def _sc_gather(weight, idx_t):
    nrows, dim = weight.shape
    sl, nb = idx_t.shape

    mesh = plsc.VectorSubcoreMesh(core_axis_name="c", subcore_axis_name="s")

    @functools.partial(
        pl.kernel,
        out_type=jax.ShapeDtypeStruct((sl, dim, nb), jnp.float32),
        mesh=mesh,
        scratch_types=[
            pltpu.VMEM((sl, _L), jnp.int32),
            pltpu.VMEM((2, _L, dim), jnp.float32),
            pltpu.VMEM((2, 1, dim, _L), jnp.float32),
            pltpu.SemaphoreType.DMA((2,)),
            pltpu.SemaphoreType.DMA((2,)),
        ],
        compiler_params=pltpu.CompilerParams(use_tc_tiling_on_sc=True, needs_layout_passes=False),
    )
    def k(table_hbm, idx_hbm, out_hbm, idx_v, rows_v, blk_v, gsem, wsem):
        wid = lax.axis_index("s") * _NC + lax.axis_index("c")
        lane0 = pl.multiple_of(wid * _L, _L)

        pltpu.sync_copy(idx_hbm.at[:, pl.ds(lane0, _L)], idx_v)

        def fire_rows(s, p):
            vecs = [idx_v.at[s][pl.ds(16 * g, 16)] for g in range(_L // 16)]
            for b in range(_L):
                iv = vecs[b // 16][b % 16]
                pltpu.async_copy(
                    table_hbm.at[pl.ds(iv, 1)],
                    rows_v.at[p].at[pl.ds(b, 1)],
                    gsem.at[p],
                )

        def drain_rows(p):
            pltpu.make_async_copy(
                table_hbm.at[pl.ds(0, _L)], rows_v.at[p], gsem.at[p]
            ).wait()

        def transpose(p):
            src = rows_v.at[p]
            dst = blk_v.at[p, 0]
            for g in range(_L // 16):
                bvec = jax.lax.iota(jnp.int32, 16) + 16 * g
                for d in range(dim):
                    dvec = jnp.full((16,), d, jnp.int32)
                    col = plsc.load_gather(src, [bvec, dvec])
                    dst[d, pl.ds(16 * g, 16)] = col

        def fire_write(s, p):
            pltpu.async_copy(
                blk_v.at[p],
                out_hbm.at[pl.ds(s, 1), :, pl.ds(lane0, _L)],
                wsem.at[p],
            )

        def drain_write(s, p):
            pltpu.make_async_copy(
                blk_v.at[p],
                out_hbm.at[pl.ds(s, 1), :, pl.ds(lane0, _L)],
                wsem.at[p],
            ).wait()

        fire_rows(0, 0)

        @pl.loop(1, sl)
        def _(s):
            p = lax.rem(s, 2)
            @pl.when(s >= 3)
            def _():
                drain_write(s - 3, 1 - p)
            fire_rows(s, p)
            drain_rows(1 - p)
            transpose(1 - p)
            fire_write(s - 1, 1 - p)

        pf = lax.rem(sl - 1, 2)
        drain_rows(pf)
        drain_write(sl - 3, pf)
        transpose(pf)
        fire_write(sl - 1, pf)
        drain_write(sl - 2, 1 - pf)
        drain_write(sl - 1, pf)

    return k(weight, idx_t)


def kernel(idx, weight):
    out_t = _sc_gather(_tc_transpose(weight.T), idx.T)
    return jnp.transpose(out_t, (2, 0, 1))


# speedup vs baseline: 1.5763x; 1.0005x over previous
import functools

import jax
import jax.numpy as jnp
from jax import lax
from jax.experimental import pallas as pl
from jax.experimental.pallas import tpu as pltpu
from jax.experimental.pallas import tpu_sc as plsc

_info = plsc.get_sparse_core_info()
_NC, _NS = _info.num_cores, _info.num_subcores
_NW = _NC * _NS
_L = 128


_TBLK = 49152


@jax.jit
def _tc_transpose(weight_t):
    dim, nrows = weight_t.shape
    grid = (nrows + _TBLK - 1) // _TBLK

    def body(in_ref, out_ref):
        out_ref[...] = in_ref[...].T

    return pl.pallas_call(
        body,
        grid=(grid,),
        in_specs=[pl.BlockSpec((dim, _TBLK), lambda i: (0, i))],
        out_specs=pl.BlockSpec((_TBLK, dim), lambda i: (i, 0)),
        out_shape=jax.ShapeDtypeStruct((nrows, dim), jnp.float32),
        compiler_params=pltpu.CompilerParams(vmem_limit_bytes=63 * 1024 * 1024),
    )(weight_t)


@jax.jit
def _sc_gather(weight, idx_t):
    nrows, dim = weight.shape
    sl, nb = idx_t.shape

    mesh = plsc.VectorSubcoreMesh(core_axis_name="c", subcore_axis_name="s")

    @functools.partial(
        pl.kernel,
        out_type=jax.ShapeDtypeStruct((sl, dim, nb), jnp.float32),
        mesh=mesh,
        scratch_types=[
            pltpu.VMEM((sl, _L), jnp.int32),
            pltpu.VMEM((2, _L, dim), jnp.float32),
            pltpu.VMEM((2, 1, dim, _L), jnp.float32),
            pltpu.SemaphoreType.DMA((2,)),
            pltpu.SemaphoreType.DMA((2,)),
        ],
        compiler_params=pltpu.CompilerParams(use_tc_tiling_on_sc=True, needs_layout_passes=False),
    )
    def k(table_hbm, idx_hbm, out_hbm, idx_v, rows_v, blk_v, gsem, wsem):
        wid = lax.axis_index("s") * _NC + lax.axis_index("c")
        lane0 = pl.multiple_of(wid * _L, _L)

        pltpu.sync_copy(idx_hbm.at[:, pl.ds(lane0, _L)], idx_v)

        def fire_rows(s, p):
            vecs = [idx_v.at[s][pl.ds(16 * g, 16)] for g in range(_L // 16)]
            for b in range(_L):
                iv = vecs[b // 16][b % 16]
                pltpu.async_copy(
                    table_hbm.at[pl.ds(iv, 1)],
                    rows_v.at[p].at[pl.ds(b, 1)],
                    gsem.at[p],
                )

        def drain_rows(p):
            pltpu.make_async_copy(
                table_hbm.at[pl.ds(0, _L)], rows_v.at[p], gsem.at[p]
            ).wait()

        def transpose(p):
            src = rows_v.at[p]
            dst = blk_v.at[p, 0]
            for g in range(_L // 16):
                bvec = jax.lax.iota(jnp.int32, 16) + 16 * g
                for d in range(dim):
                    dvec = jnp.full((16,), d, jnp.int32)
                    col = plsc.load_gather(src, [bvec, dvec])
                    dst[d, pl.ds(16 * g, 16)] = col

        def fire_write(s, p):
            pltpu.async_copy(
                blk_v.at[p],
                out_hbm.at[pl.ds(s, 1), :, pl.ds(lane0, _L)],
                wsem.at[p],
            )

        def drain_write(s, p):
            pltpu.make_async_copy(
                blk_v.at[p],
                out_hbm.at[pl.ds(s, 1), :, pl.ds(lane0, _L)],
                wsem.at[p],
            ).wait()

        fire_rows(0, 0)

        @pl.loop(1, sl)
        def _(s):
            p = lax.rem(s, 2)
            @pl.when(s >= 3)
            def _():
                drain_write(s - 3, 1 - p)
            fire_rows(s, p)
            drain_rows(1 - p)
            transpose(1 - p)
            fire_write(s - 1, 1 - p)

        pf = lax.rem(sl - 1, 2)
        drain_rows(pf)
        drain_write(sl - 3, pf)
        transpose(pf)
        fire_write(sl - 1, pf)
        drain_write(sl - 2, 1 - pf)
        drain_write(sl - 1, pf)

    return k(weight, idx_t)


def kernel(idx, weight):
    out_t = _sc_gather(_tc_transpose(weight.T), idx.T)
    return jnp.transpose(out_t, (2, 0, 1))
